# Initial kernel scaffold; baseline (speedup 1.0000x reference)
import jax
import jax.numpy as jnp
from jax.experimental import pallas as pl


def kernel(inputs, summ0_indices, summ0_values, summ1_indices, summ1_values, dag_W0, dag_b0, dag_W1, dag_b1, dag_W2, dag_b2, glob_W0, glob_b0, glob_W1, glob_b1, glob_W2, glob_b2):
    raise NotImplementedError("write your pallas kernel here")


# speedup vs baseline: 4.4232x; 4.4232x over previous
import functools

import jax
import jax.numpy as jnp
from jax import lax
from jax.experimental import pallas as pl
from jax.experimental.pallas import tpu as pltpu
from jax.experimental.pallas import tpu_sc as plsc

N = 10000
D = 128
NC = 2
NS = 16
CHUNK = 128
LANES = 16


_BLK = 1000


def _mlp_body(h, wbs):
    for w, b in wbs:
        h = jnp.maximum(jnp.dot(h, w[...], preferred_element_type=jnp.float32)
                        + b[...], 0.0)
    return h


def _mlp3_tc(x, W0, b0, W1, b1, W2, b2):
    grid = (N // _BLK,)
    wspec = pl.BlockSpec((D, D), lambda i: (0, 0))
    bspec = pl.BlockSpec((1, D), lambda i: (0, 0))

    def body(x_ref, w0, b0r, w1, b1r, w2, b2r, o_ref):
        o_ref[...] = _mlp_body(x_ref[...], ((w0, b0r), (w1, b1r), (w2, b2r)))

    return pl.pallas_call(
        body,
        grid=grid,
        in_specs=[pl.BlockSpec((_BLK, D), lambda i: (i, 0)),
                  wspec, bspec, wspec, bspec, wspec, bspec],
        out_specs=pl.BlockSpec((_BLK, D), lambda i: (i, 0)),
        out_shape=jax.ShapeDtypeStruct((N, D), jnp.float32),
    )(x, W0, b0.reshape(1, D), W1, b1.reshape(1, D), W2, b2.reshape(1, D))


def _combine_mlp3_tc(parts, W0, b0, W1, b1, W2, b2):
    grid = (N // _BLK,)
    wspec = pl.BlockSpec((D, D), lambda i: (0, 0))
    bspec = pl.BlockSpec((1, D), lambda i: (0, 0))

    def body(p0_ref, p1_ref, w0, b0r, w1, b1r, w2, b2r, s_ref, h_ref):
        x = p0_ref[0] + p1_ref[0]
        s_ref[...] = x
        h_ref[...] = _mlp_body(x, ((w0, b0r), (w1, b1r), (w2, b2r)))

    return pl.pallas_call(
        body,
        grid=grid,
        in_specs=[pl.BlockSpec((1, _BLK, D), lambda i: (0, i, 0)),
                  pl.BlockSpec((1, _BLK, D), lambda i: (1, i, 0)),
                  wspec, bspec, wspec, bspec, wspec, bspec],
        out_specs=[pl.BlockSpec((_BLK, D), lambda i: (i, 0)),
                   pl.BlockSpec((_BLK, D), lambda i: (i, 0))],
        out_shape=[jax.ShapeDtypeStruct((N, D), jnp.float32),
                   jax.ShapeDtypeStruct((N, D), jnp.float32)],
    )(parts, parts, W0, b0.reshape(1, D), W1, b1.reshape(1, D),
      W2, b2.reshape(1, D))


def _combine_tc(parts):
    grid = (N // _BLK,)

    def body(p0_ref, p1_ref, o_ref):
        o_ref[...] = p0_ref[0] + p1_ref[0]

    return pl.pallas_call(
        body,
        grid=grid,
        in_specs=[pl.BlockSpec((1, _BLK, D), lambda i: (0, i, 0)),
                  pl.BlockSpec((1, _BLK, D), lambda i: (1, i, 0))],
        out_specs=pl.BlockSpec((_BLK, D), lambda i: (i, 0)),
        out_shape=jax.ShapeDtypeStruct((N, D), jnp.float32),
    )(parts, parts)


def _prep_edges(indices, values):
    e = values.shape[0]
    per_tile = -(-e // (NC * NS * CHUNK)) * CHUNK
    epad = per_tile * NC * NS
    pad = epad - e
    row = jnp.concatenate([indices[0], jnp.zeros((pad,), jnp.int32)])
    col = jnp.concatenate([indices[1], jnp.zeros((pad,), jnp.int32)])
    val = jnp.concatenate([values, jnp.zeros((pad,), jnp.float32)])
    shape = (NC, NS, per_tile // CHUNK, CHUNK)
    return row.reshape(shape), col.reshape(shape), val.reshape(shape)


def _spmm_sc(h, row, col, val):
    nchunks = row.shape[2]
    rpt = (N // NS) // 8 * 8
    tail = N - NS * rpt
    nz_full = rpt // CHUNK
    nz_rem = rpt % CHUNK
    mesh = plsc.VectorSubcoreMesh(core_axis_name="c", subcore_axis_name="s")

    @functools.partial(
        pl.kernel,
        out_type=jax.ShapeDtypeStruct((NC, N, D), jnp.float32),
        mesh=mesh,
        scratch_types=[
            pltpu.VMEM((nchunks, CHUNK), jnp.int32),
            pltpu.VMEM((nchunks, CHUNK), jnp.int32),
            pltpu.VMEM((nchunks, CHUNK), jnp.float32),
            pltpu.VMEM((CHUNK, D), jnp.float32),
            pltpu.VMEM_SHARED((N, D), jnp.float32),
            pltpu.SemaphoreType.DMA,
        ],
    )
    def k(h_hbm, row_hbm, col_hbm, val_hbm, out_hbm,
          row_v, col_v, val_v, gbuf, acc, gsem):
        c = lax.axis_index("c")
        s = lax.axis_index("s")

        pltpu.sync_copy(row_hbm.at[c, s], row_v)
        pltpu.sync_copy(col_hbm.at[c, s], col_v)
        pltpu.sync_copy(val_hbm.at[c, s], val_v)

        zero = jnp.zeros((LANES,), jnp.float32)

        def zbody(r, _):
            for q in range(D // LANES):
                gbuf[r, pl.ds(q * LANES, LANES)] = zero
            return 0

        lax.fori_loop(0, CHUNK, zbody, 0)
        base = s * rpt
        for t in range(nz_full):
            pltpu.sync_copy(gbuf, acc.at[pl.ds(base + t * CHUNK, CHUNK)])
        if nz_rem:
            pltpu.sync_copy(gbuf.at[pl.ds(0, nz_rem)],
                            acc.at[pl.ds(base + nz_full * CHUNK, nz_rem)])
        if tail:
            @pl.when(s == NS - 1)
            def _():
                pltpu.sync_copy(gbuf.at[pl.ds(0, tail)],
                                acc.at[pl.ds(NS * rpt, tail)])
        plsc.subcore_barrier()

        def chunk_body(j, _):
            pltpu.async_copy(h_hbm.at[col_v.at[j]], gbuf, gsem).wait()

            dnums = lax.GatherDimensionNumbers(
                offset_dims=(), collapsed_slice_dims=(0,), start_index_map=(0,))

            def gbody(g, _):
                vv = val_v[j, pl.ds(g * LANES, LANES)]
                for i in range(LANES):
                    b = lax.gather(
                        vv, jnp.full((LANES, 1), i, jnp.int32), dnums,
                        slice_sizes=(1,),
                        mode=lax.GatherScatterMode.PROMISE_IN_BOUNDS)
                    e = g * LANES + i
                    for q in range(D // LANES):
                        sl = pl.ds(q * LANES, LANES)
                        gbuf[e, sl] = gbuf[e, sl] * b
                return 0

            lax.fori_loop(0, CHUNK // LANES, gbody, 0)
            pltpu.sync_copy(gbuf, acc.at[row_v.at[j]], add=True)
            return 0

        lax.fori_loop(0, nchunks, chunk_body, 0)
        plsc.subcore_barrier()

        pltpu.sync_copy(acc.at[pl.ds(base, rpt)], out_hbm.at[c, pl.ds(base, rpt)])
        if tail:
            @pl.when(s == NS - 1)
            def _():
                pltpu.sync_copy(acc.at[pl.ds(NS * rpt, tail)],
                                out_hbm.at[c, pl.ds(NS * rpt, tail)])

    return k(h, row, col, val)


def kernel(inputs, summ0_indices, summ0_values, summ1_indices, summ1_values,
           dag_W0, dag_b0, dag_W1, dag_b1, dag_W2, dag_b2,
           glob_W0, glob_b0, glob_W1, glob_b1, glob_W2, glob_b2):
    h0 = _mlp3_tc(inputs, dag_W0, dag_b0, dag_W1, dag_b1, dag_W2, dag_b2)
    r0, c0, v0 = _prep_edges(summ0_indices, summ0_values)
    parts0 = _spmm_sc(h0, r0, c0, v0)
    out0, h1 = _combine_mlp3_tc(parts0, glob_W0, glob_b0, glob_W1, glob_b1,
                                glob_W2, glob_b2)
    r1, c1, v1 = _prep_edges(summ1_indices, summ1_values)
    parts1 = _spmm_sc(h1, r1, c1, v1)
    out1 = _combine_tc(parts1)
    return (out0, out1)
